# Initial kernel scaffold; baseline (speedup 1.0000x reference)
import jax
import jax.numpy as jnp
from jax.experimental import pallas as pl


def kernel(input, target):
    raise NotImplementedError("write your pallas kernel here")


# speedup vs baseline: 1.0585x; 1.0585x over previous
import jax
import jax.numpy as jnp
from jax import lax
from jax.experimental import pallas as pl
from jax.experimental.pallas import tpu as pltpu
from jax.experimental.pallas import tpu_sc as plsc

_H = 512
_W = 512
_B = 16384
_NC = 2
_NS = 16
_NW = _NC * _NS
_BPW = _B // _NW
_VEC = 16
_CHUNK = 128
_NCHUNK = _BPW // _CHUNK

_mesh = plsc.VectorSubcoreMesh(
    core_axis_name="c", subcore_axis_name="s", num_cores=_NC, num_subcores=_NS)
_params = pltpu.CompilerParams(needs_layout_passes=False)


def _partials_body(img, x1c, y1c, x2c, y2c, wc, labc, out,
                   x1_v, y1_v, x2_v, y2_v, w_v, lab_v, idx_v, val_v, row_v,
                   sem):
    wid = lax.axis_index("s") * _NC + lax.axis_index("c")
    base = wid * _BPW

    pltpu.sync_copy(x1c.at[pl.ds(base, _BPW)], x1_v)
    pltpu.sync_copy(y1c.at[pl.ds(base, _BPW)], y1_v)
    pltpu.sync_copy(x2c.at[pl.ds(base, _BPW)], x2_v)
    pltpu.sync_copy(y2c.at[pl.ds(base, _BPW)], y2_v)
    pltpu.sync_copy(wc.at[pl.ds(base, _BPW)], w_v)
    pltpu.sync_copy(labc.at[pl.ds(base, _BPW)], lab_v)

    for i in range(_BPW // _VEC):
        sl = pl.ds(i * _VEC, _VEC)
        ix1 = (x1_v[sl] * float(_W)).astype(jnp.int32)
        iy1 = (y1_v[sl] * float(_H)).astype(jnp.int32)
        ix2 = (x2_v[sl] * float(_W)).astype(jnp.int32)
        iy2 = (y2_v[sl] * float(_H)).astype(jnp.int32)
        r, c0 = i // (_CHUNK // _VEC), (i % (_CHUNK // _VEC)) * _VEC
        idx_v[r, pl.ds(c0, _VEC)] = iy1 * _W + ix1
        idx_v[_NCHUNK + r, pl.ds(c0, _VEC)] = iy2 * _W + ix2

    cps = [pltpu.async_copy(img.at[idx_v.at[j]], val_v.at[j], sem)
           for j in range(2 * _NCHUNK)]
    for cp in cps:
        cp.wait()

    acc_whdr = jnp.zeros((_VEC,), jnp.float32)
    acc_wt = jnp.zeros((_VEC,), jnp.float32)
    for i in range(_BPW // _VEC):
        r, c0 = i // (_CHUNK // _VEC), (i % (_CHUNK // _VEC)) * _VEC
        slv = pl.ds(c0, _VEC)
        sl = pl.ds(i * _VEC, _VEC)
        divisor = val_v[r, slv]
        dividend = val_v[_NCHUNK + r, slv]
        ratio = divisor / (dividend + 1e-07)
        p = jnp.where(ratio <= 1.0 + 0.12, ratio, 2.0)
        p = jnp.where(p >= 1.0 / (1.0 + 0.12), p, 1.0)
        p = jnp.where(p == 1.0, p, 0.0) + jnp.where(p == 2.0, p, 0.0)
        wv = w_v[sl]
        acc_whdr = acc_whdr + jnp.where(lab_v[sl] != p, wv, 0.0)
        acc_wt = acc_wt + wv

    row_v[pl.ds(0, _VEC)] = acc_whdr
    row_v[pl.ds(_VEC, _VEC)] = acc_wt
    pltpu.sync_copy(row_v, out.at[wid])


def _lane_sum_broadcast(vec, buf_ref):
    buf_ref[pl.ds(_VEC, _VEC)] = jnp.zeros((_VEC,), jnp.float32)
    cur = vec
    for s in (8, 4, 2, 1):
        buf_ref[pl.ds(0, _VEC)] = cur
        cur = cur + buf_ref[pl.ds(s, _VEC)]
    buf_ref[pl.ds(0, _VEC)] = cur
    zeros_i = jnp.zeros((_VEC,), jnp.int32)
    return plsc.load_gather(buf_ref, [zeros_i])


def _finalize_body(parts, out, acc_v, buf_v, out_v):
    wid = lax.axis_index("s") * _NC + lax.axis_index("c")

    @pl.when(wid == 0)
    def _():
        pltpu.sync_copy(parts, acc_v)
        whdr_acc = jnp.zeros((_VEC,), jnp.float32)
        wt_acc = jnp.zeros((_VEC,), jnp.float32)
        for i in range(_NW):
            whdr_acc = whdr_acc + acc_v[i, pl.ds(0, _VEC)]
            wt_acc = wt_acc + acc_v[i, pl.ds(_VEC, _VEC)]
        whdr_tot = _lane_sum_broadcast(whdr_acc, buf_v)
        wt_tot = _lane_sum_broadcast(wt_acc, buf_v)
        out_v[...] = whdr_tot / wt_tot
        pltpu.sync_copy(out_v, out)


def kernel(input, target):
    img = input.reshape(_H * _W)
    t = target[0]
    parts = pl.kernel(
        _partials_body,
        out_type=jax.ShapeDtypeStruct((_NW, 2 * _VEC), jnp.float32),
        mesh=_mesh,
        compiler_params=_params,
        scratch_types=[
            pltpu.VMEM((_BPW,), jnp.float32),
            pltpu.VMEM((_BPW,), jnp.float32),
            pltpu.VMEM((_BPW,), jnp.float32),
            pltpu.VMEM((_BPW,), jnp.float32),
            pltpu.VMEM((_BPW,), jnp.float32),
            pltpu.VMEM((_BPW,), jnp.float32),
            pltpu.VMEM((2 * _NCHUNK, _CHUNK), jnp.int32),
            pltpu.VMEM((2 * _NCHUNK, _CHUNK), jnp.float32),
            pltpu.VMEM((2 * _VEC,), jnp.float32),
            pltpu.SemaphoreType.DMA,
        ],
    )(img, t[:, 2], t[:, 3], t[:, 4], t[:, 5], t[:, 0], t[:, 1])
    res = pl.kernel(
        _finalize_body,
        out_type=jax.ShapeDtypeStruct((_VEC,), jnp.float32),
        mesh=_mesh,
        compiler_params=_params,
        scratch_types=[
            pltpu.VMEM((_NW, 2 * _VEC), jnp.float32),
            pltpu.VMEM((2 * _VEC,), jnp.float32),
            pltpu.VMEM((_VEC,), jnp.float32),
        ],
    )(parts)
    return res[0]


# speedup vs baseline: 1.0689x; 1.0099x over previous
import jax
import jax.numpy as jnp
from jax import lax
from jax.experimental import pallas as pl
from jax.experimental.pallas import tpu as pltpu
from jax.experimental.pallas import tpu_sc as plsc

_H = 512
_W = 512
_B = 16384
_NC = 2
_NS = 16
_NW = _NC * _NS
_BPW = _B // _NW
_VEC = 16
_CHUNK = 128
_NCHUNK = _BPW // _CHUNK
_SPC = _CHUNK // _VEC

_mesh = plsc.VectorSubcoreMesh(
    core_axis_name="c", subcore_axis_name="s", num_cores=_NC, num_subcores=_NS)
_params = pltpu.CompilerParams(needs_layout_passes=False)


def _lane_sum_broadcast(vec, buf_ref):
    buf_ref[pl.ds(_VEC, _VEC)] = jnp.zeros((_VEC,), jnp.float32)
    cur = vec
    for s in (8, 4, 2, 1):
        buf_ref[pl.ds(0, _VEC)] = cur
        cur = cur + buf_ref[pl.ds(s, _VEC)]
    buf_ref[pl.ds(0, _VEC)] = cur
    zeros_i = jnp.zeros((_VEC,), jnp.int32)
    return plsc.load_gather(buf_ref, [zeros_i])


def _body(img, x1c, y1c, x2c, y2c, wc, labc, out,
          x1_v, y1_v, x2_v, y2_v, w_v, lab_v, idx_v, val_v, red_v, gath_v,
          buf_v, row_v, shared_v, sems):
    cid = lax.axis_index("c")
    sid = lax.axis_index("s")
    wid = sid * _NC + cid
    base = wid * _BPW

    pltpu.sync_copy(x1c.at[pl.ds(base, _BPW)], x1_v)
    pltpu.sync_copy(y1c.at[pl.ds(base, _BPW)], y1_v)
    pltpu.sync_copy(x2c.at[pl.ds(base, _BPW)], x2_v)
    pltpu.sync_copy(y2c.at[pl.ds(base, _BPW)], y2_v)
    pltpu.sync_copy(wc.at[pl.ds(base, _BPW)], w_v)
    pltpu.sync_copy(labc.at[pl.ds(base, _BPW)], lab_v)

    lanes = lax.iota(jnp.int32, _VEC)

    cps = []
    for r in range(_NCHUNK):
        for k in range(_SPC):
            sl = pl.ds((r * _SPC + k) * _VEC, _VEC)
            ix1 = (x1_v[sl] * float(_W)).astype(jnp.int32)
            iy1 = (y1_v[sl] * float(_H)).astype(jnp.int32)
            ix2 = (x2_v[sl] * float(_W)).astype(jnp.int32)
            iy2 = (y2_v[sl] * float(_H)).astype(jnp.int32)
            idx_v[r, pl.ds(k * _VEC, _VEC)] = iy1 * _W + ix1
            idx_v[_NCHUNK + r, pl.ds(k * _VEC, _VEC)] = iy2 * _W + ix2
        cps.append((
            pltpu.async_copy(img.at[idx_v.at[r]], val_v.at[r], sems.at[r]),
            pltpu.async_copy(img.at[idx_v.at[_NCHUNK + r]],
                             val_v.at[_NCHUNK + r], sems.at[_NCHUNK + r]),
        ))

    acc_whdr = jnp.zeros((_VEC,), jnp.float32)
    acc_wt = jnp.zeros((_VEC,), jnp.float32)
    for r in range(_NCHUNK):
        cps[r][0].wait()
        cps[r][1].wait()
        for k in range(_SPC):
            slv = pl.ds(k * _VEC, _VEC)
            sl = pl.ds((r * _SPC + k) * _VEC, _VEC)
            divisor = val_v[r, slv]
            dividend = val_v[_NCHUNK + r, slv]
            ratio = divisor / (dividend + 1e-07)
            p = jnp.where(ratio <= 1.0 + 0.12, ratio, 2.0)
            p = jnp.where(p >= 1.0 / (1.0 + 0.12), p, 1.0)
            p = jnp.where(p == 1.0, p, 0.0) + jnp.where(p == 2.0, p, 0.0)
            wv = w_v[sl]
            acc_whdr = acc_whdr + jnp.where(lab_v[sl] != p, wv, 0.0)
            acc_wt = acc_wt + wv

    red_v[pl.ds(0, _VEC)] = acc_whdr
    red_v[pl.ds(_VEC, _VEC)] = acc_wt
    pltpu.sync_copy(red_v, shared_v.at[sid])
    plsc.subcore_barrier()

    @pl.when(sid == 0)
    def _():
        pltpu.sync_copy(shared_v, gath_v)
        whdr_acc = jnp.zeros((_VEC,), jnp.float32)
        wt_acc = jnp.zeros((_VEC,), jnp.float32)
        for i in range(_NS):
            whdr_acc = whdr_acc + gath_v[i, pl.ds(0, _VEC)]
            wt_acc = wt_acc + gath_v[i, pl.ds(_VEC, _VEC)]
        whdr_tot = _lane_sum_broadcast(whdr_acc, buf_v)
        wt_tot = _lane_sum_broadcast(wt_acc, buf_v)
        row_v[...] = (jnp.where(lanes == 0, whdr_tot, 0.0)
                      + jnp.where(lanes == 1, wt_tot, 0.0))
        pltpu.sync_copy(row_v, out.at[cid])


def kernel(input, target):
    img = input.reshape(_H * _W)
    t = target[0]
    parts = pl.kernel(
        _body,
        out_type=jax.ShapeDtypeStruct((_NC, _VEC), jnp.float32),
        mesh=_mesh,
        compiler_params=_params,
        scratch_types=[
            pltpu.VMEM((_BPW,), jnp.float32),
            pltpu.VMEM((_BPW,), jnp.float32),
            pltpu.VMEM((_BPW,), jnp.float32),
            pltpu.VMEM((_BPW,), jnp.float32),
            pltpu.VMEM((_BPW,), jnp.float32),
            pltpu.VMEM((_BPW,), jnp.float32),
            pltpu.VMEM((2 * _NCHUNK, _CHUNK), jnp.int32),
            pltpu.VMEM((2 * _NCHUNK, _CHUNK), jnp.float32),
            pltpu.VMEM((2 * _VEC,), jnp.float32),
            pltpu.VMEM((_NS, 2 * _VEC), jnp.float32),
            pltpu.VMEM((2 * _VEC,), jnp.float32),
            pltpu.VMEM((_VEC,), jnp.float32),
            pltpu.VMEM_SHARED((_NS, 2 * _VEC), jnp.float32),
            pltpu.SemaphoreType.DMA((2 * _NCHUNK,)),
        ],
    )(img, t[:, 2], t[:, 3], t[:, 4], t[:, 5], t[:, 0], t[:, 1])
    return (parts[0, 0] + parts[1, 0]) / (parts[0, 1] + parts[1, 1])


# speedup vs baseline: 1.1547x; 1.0803x over previous
import jax
import jax.numpy as jnp
from jax import lax
from jax.experimental import pallas as pl
from jax.experimental.pallas import tpu as pltpu
from jax.experimental.pallas import tpu_sc as plsc

_H = 512
_W = 512
_B = 16384
_NC = 2
_NS = 16
_NW = _NC * _NS
_BPW = _B // _NW
_VEC = 16
_CHUNK = 128
_NCHUNK = _BPW // _CHUNK
_SPC = _CHUNK // _VEC

_mesh = plsc.VectorSubcoreMesh(
    core_axis_name="c", subcore_axis_name="s", num_cores=_NC, num_subcores=_NS)
_params = pltpu.CompilerParams(
    needs_layout_passes=False,
    disable_bounds_checks=True,
    disable_semaphore_checks=True,
)


def _lane_sum_broadcast(vec, buf_ref):
    buf_ref[pl.ds(_VEC, _VEC)] = jnp.zeros((_VEC,), jnp.float32)
    cur = vec
    for s in (8, 4, 2, 1):
        buf_ref[pl.ds(0, _VEC)] = cur
        cur = cur + buf_ref[pl.ds(s, _VEC)]
    buf_ref[pl.ds(0, _VEC)] = cur
    zeros_i = jnp.zeros((_VEC,), jnp.int32)
    return plsc.load_gather(buf_ref, [zeros_i])


def _body(img, x1c, y1c, x2c, y2c, wc, labc, out,
          x1_v, y1_v, x2_v, y2_v, w_v, lab_v, idx_v, val_v, red_v, gath_v,
          buf_v, row_v, shared_v, sems, in_sems):
    cid = lax.axis_index("c")
    sid = lax.axis_index("s")
    wid = sid * _NC + cid
    base = wid * _BPW

    in_cps = [
        pltpu.async_copy(src.at[pl.ds(base, _BPW)], dst,
                         in_sems.at[i])
        for i, (src, dst) in enumerate([
            (x1c, x1_v), (y1c, y1_v), (x2c, x2_v), (y2c, y2_v),
            (wc, w_v), (labc, lab_v)])
    ]
    for cp in in_cps[:4]:
        cp.wait()

    lanes = lax.iota(jnp.int32, _VEC)

    cps = []
    for r in range(_NCHUNK):
        for k in range(_SPC):
            sl = pl.ds((r * _SPC + k) * _VEC, _VEC)
            ix1 = (x1_v[sl] * float(_W)).astype(jnp.int32)
            iy1 = (y1_v[sl] * float(_H)).astype(jnp.int32)
            ix2 = (x2_v[sl] * float(_W)).astype(jnp.int32)
            iy2 = (y2_v[sl] * float(_H)).astype(jnp.int32)
            idx_v[r, pl.ds(k * _VEC, _VEC)] = iy1 * _W + ix1
            idx_v[_NCHUNK + r, pl.ds(k * _VEC, _VEC)] = iy2 * _W + ix2
        cps.append((
            pltpu.async_copy(img.at[idx_v.at[r]], val_v.at[r], sems.at[r]),
            pltpu.async_copy(img.at[idx_v.at[_NCHUNK + r]],
                             val_v.at[_NCHUNK + r], sems.at[_NCHUNK + r]),
        ))

    in_cps[4].wait()
    in_cps[5].wait()
    acc_whdr = jnp.zeros((_VEC,), jnp.float32)
    acc_wt = jnp.zeros((_VEC,), jnp.float32)
    for r in range(_NCHUNK):
        cps[r][0].wait()
        cps[r][1].wait()
        for k in range(_SPC):
            slv = pl.ds(k * _VEC, _VEC)
            sl = pl.ds((r * _SPC + k) * _VEC, _VEC)
            divisor = val_v[r, slv]
            dividend = val_v[_NCHUNK + r, slv]
            ratio = divisor / (dividend + 1e-07)
            p = jnp.where(ratio <= 1.0 + 0.12, ratio, 2.0)
            p = jnp.where(p >= 1.0 / (1.0 + 0.12), p, 1.0)
            p = jnp.where(p == 1.0, p, 0.0) + jnp.where(p == 2.0, p, 0.0)
            wv = w_v[sl]
            acc_whdr = acc_whdr + jnp.where(lab_v[sl] != p, wv, 0.0)
            acc_wt = acc_wt + wv

    red_v[pl.ds(0, _VEC)] = acc_whdr
    red_v[pl.ds(_VEC, _VEC)] = acc_wt
    pltpu.sync_copy(red_v, shared_v.at[sid])
    plsc.subcore_barrier()

    @pl.when(sid == 0)
    def _():
        pltpu.sync_copy(shared_v, gath_v)
        whdr_acc = jnp.zeros((_VEC,), jnp.float32)
        wt_acc = jnp.zeros((_VEC,), jnp.float32)
        for i in range(_NS):
            whdr_acc = whdr_acc + gath_v[i, pl.ds(0, _VEC)]
            wt_acc = wt_acc + gath_v[i, pl.ds(_VEC, _VEC)]
        whdr_tot = _lane_sum_broadcast(whdr_acc, buf_v)
        wt_tot = _lane_sum_broadcast(wt_acc, buf_v)
        row_v[...] = (jnp.where(lanes == 0, whdr_tot, 0.0)
                      + jnp.where(lanes == 1, wt_tot, 0.0))
        pltpu.sync_copy(row_v, out.at[cid])


def kernel(input, target):
    img = input.reshape(_H * _W)
    t = target[0]
    parts = pl.kernel(
        _body,
        out_type=jax.ShapeDtypeStruct((_NC, _VEC), jnp.float32),
        mesh=_mesh,
        compiler_params=_params,
        scratch_types=[
            pltpu.VMEM((_BPW,), jnp.float32),
            pltpu.VMEM((_BPW,), jnp.float32),
            pltpu.VMEM((_BPW,), jnp.float32),
            pltpu.VMEM((_BPW,), jnp.float32),
            pltpu.VMEM((_BPW,), jnp.float32),
            pltpu.VMEM((_BPW,), jnp.float32),
            pltpu.VMEM((2 * _NCHUNK, _CHUNK), jnp.int32),
            pltpu.VMEM((2 * _NCHUNK, _CHUNK), jnp.float32),
            pltpu.VMEM((2 * _VEC,), jnp.float32),
            pltpu.VMEM((_NS, 2 * _VEC), jnp.float32),
            pltpu.VMEM((2 * _VEC,), jnp.float32),
            pltpu.VMEM((_VEC,), jnp.float32),
            pltpu.VMEM_SHARED((_NS, 2 * _VEC), jnp.float32),
            pltpu.SemaphoreType.DMA((2 * _NCHUNK,)),
            pltpu.SemaphoreType.DMA((6,)),
        ],
    )(img, t[:, 2], t[:, 3], t[:, 4], t[:, 5], t[:, 0], t[:, 1])
    return (parts[0, 0] + parts[1, 0]) / (parts[0, 1] + parts[1, 1])


# speedup vs baseline: 1.2173x; 1.0542x over previous
import jax
import jax.numpy as jnp
from jax import lax
from jax.experimental import pallas as pl
from jax.experimental.pallas import tpu as pltpu
from jax.experimental.pallas import tpu_sc as plsc

_H = 512
_W = 512
_B = 16384
_NC = 2
_NS = 16
_NW = _NC * _NS
_BPW = _B // _NW
_VEC = 16
_CHUNK = 128
_NCHUNK = _BPW // _CHUNK
_SPC = _CHUNK // _VEC

_mesh = plsc.VectorSubcoreMesh(
    core_axis_name="c", subcore_axis_name="s", num_cores=_NC, num_subcores=_NS)
_params = pltpu.CompilerParams(needs_layout_passes=False)


def _lane_sum_broadcast(vec):
    lanes = lax.iota(jnp.int32, _VEC)
    cur = vec
    for s in (8, 4, 2, 1):
        perm = jnp.bitwise_xor(lanes, s)
        cur = cur + jnp.take_along_axis(cur, perm, axis=0,
                                        mode="promise_in_bounds")
    return cur


def _body(img, x1c, y1c, x2c, y2c, wc, labc, out,
          x1_v, y1_v, x2_v, y2_v, w_v, lab_v, idx_v, val_v, row_v,
          sem_g0, sem_g1, sem_g2, sem_g3, sem_in):
    cid = lax.axis_index("c")
    sid = lax.axis_index("s")
    wid = sid * _NC + cid
    base = wid * _BPW

    in_cps = [
        pltpu.async_copy(src.at[pl.ds(base, _BPW)], dst, sem_in)
        for src, dst in [
            (x1c, x1_v), (y1c, y1_v), (x2c, x2_v), (y2c, y2_v),
            (wc, w_v), (labc, lab_v)]
    ]
    for cp in in_cps:
        cp.wait()

    lanes = lax.iota(jnp.int32, _VEC)

    cps = []
    for r in range(_NCHUNK):
        for k in range(_SPC):
            sl = pl.ds((r * _SPC + k) * _VEC, _VEC)
            ix1 = (x1_v[sl] * float(_W)).astype(jnp.int32)
            iy1 = (y1_v[sl] * float(_H)).astype(jnp.int32)
            ix2 = (x2_v[sl] * float(_W)).astype(jnp.int32)
            iy2 = (y2_v[sl] * float(_H)).astype(jnp.int32)
            idx_v[r, pl.ds(k * _VEC, _VEC)] = iy1 * _W + ix1
            idx_v[_NCHUNK + r, pl.ds(k * _VEC, _VEC)] = iy2 * _W + ix2
        sem_r = (sem_g0, sem_g1, sem_g2, sem_g3)[r]
        cps.append((
            pltpu.async_copy(img.at[idx_v.at[r]], val_v.at[r], sem_r),
            pltpu.async_copy(img.at[idx_v.at[_NCHUNK + r]],
                             val_v.at[_NCHUNK + r], sem_r),
        ))

    acc_whdr = jnp.zeros((_VEC,), jnp.float32)
    acc_wt = jnp.zeros((_VEC,), jnp.float32)
    for r in range(_NCHUNK):
        cps[r][0].wait()
        cps[r][1].wait()
        for k in range(_SPC):
            slv = pl.ds(k * _VEC, _VEC)
            sl = pl.ds((r * _SPC + k) * _VEC, _VEC)
            divisor = val_v[r, slv]
            dividend = val_v[_NCHUNK + r, slv]
            ratio = divisor / (dividend + 1e-07)
            p = jnp.where(ratio <= 1.0 + 0.12, ratio, 2.0)
            p = jnp.where(p >= 1.0 / (1.0 + 0.12), p, 1.0)
            p = jnp.where(p == 1.0, p, 0.0) + jnp.where(p == 2.0, p, 0.0)
            wv = w_v[sl]
            acc_whdr = acc_whdr + jnp.where(lab_v[sl] != p, wv, 0.0)
            acc_wt = acc_wt + wv

    whdr_tot = _lane_sum_broadcast(acc_whdr)
    wt_tot = _lane_sum_broadcast(acc_wt)
    row_v[...] = (jnp.where(lanes == 0, whdr_tot, 0.0)
                  + jnp.where(lanes == 1, wt_tot, 0.0))
    pltpu.sync_copy(row_v, out.at[wid])


def kernel(input, target):
    img = input.reshape(_H * _W)
    t = target[0]
    parts = pl.kernel(
        _body,
        out_type=jax.ShapeDtypeStruct((_NW, _VEC), jnp.float32),
        mesh=_mesh,
        compiler_params=_params,
        scratch_types=[
            pltpu.VMEM((_BPW,), jnp.float32),
            pltpu.VMEM((_BPW,), jnp.float32),
            pltpu.VMEM((_BPW,), jnp.float32),
            pltpu.VMEM((_BPW,), jnp.float32),
            pltpu.VMEM((_BPW,), jnp.float32),
            pltpu.VMEM((_BPW,), jnp.float32),
            pltpu.VMEM((2 * _NCHUNK, _CHUNK), jnp.int32),
            pltpu.VMEM((2 * _NCHUNK, _CHUNK), jnp.float32),
            pltpu.VMEM((_VEC,), jnp.float32),
            pltpu.SemaphoreType.DMA,
            pltpu.SemaphoreType.DMA,
            pltpu.SemaphoreType.DMA,
            pltpu.SemaphoreType.DMA,
            pltpu.SemaphoreType.DMA,
        ],
    )(img, t[:, 2], t[:, 3], t[:, 4], t[:, 5], t[:, 0], t[:, 1])
    return jnp.sum(parts[:, 0]) / jnp.sum(parts[:, 1])
